# baseline (speedup 1.0000x reference)
import functools

import jax
import jax.numpy as jnp
from jax.experimental import pallas as pl
from jax.experimental.pallas import tpu as pltpu

_BM = 512


def _gmm_body(em, tm, vm, se, ee, x_ref, win_ref, wout_ref, gates_ref, out_ref):
    i = pl.program_id(0)
    e = em[i]
    t = tm[i]
    valid = vm[i]
    bm, d = x_ref.shape
    h2 = win_ref.shape[2]
    h = h2 // 2

    @pl.when(valid == 1)
    def _():
        rows = t * bm + jax.lax.broadcasted_iota(jnp.int32, (bm, 1), 0)
        mask = (rows >= se[e]) & (rows < ee[e])
        hh = jnp.dot(x_ref[...], win_ref[0], preferred_element_type=jnp.float32)
        h1 = hh[:, :h]
        hg = hh[:, h:]
        act = h1 * jax.nn.sigmoid(h1) * hg
        o = jnp.dot(act, wout_ref[0], preferred_element_type=jnp.float32)
        o = o * gates_ref[...]
        out_ref[...] = jnp.where(mask, o, out_ref[...])


def _grouped_ffn(x_sorted, w_in, w_out, gates_sorted, starts, ends):
    tk, d = x_sorted.shape
    e_num, _, h2 = w_in.shape
    h = h2 // 2
    m_tiles = tk // _BM
    max_steps = m_tiles + e_num - 1

    counts = ends - starts
    tile_lo = starts // _BM
    tile_hi = (ends + _BM - 1) // _BM
    ntiles = jnp.where(counts > 0, tile_hi - tile_lo, 0)
    cum = jnp.cumsum(ntiles)
    total = cum[-1]
    first_step = cum - ntiles

    steps = jnp.arange(max_steps, dtype=jnp.int32)
    e_of = jnp.searchsorted(cum, steps, side="right").astype(jnp.int32)
    valid = (steps < total).astype(jnp.int32)
    e_last = jnp.searchsorted(cum, total - 1, side="right").astype(jnp.int32)
    e_of = jnp.where(valid == 1, jnp.minimum(e_of, e_num - 1), e_last)
    t_of = jnp.where(
        valid == 1,
        tile_lo[e_of] + steps - first_step[e_of],
        m_tiles - 1,
    ).astype(jnp.int32)

    grid_spec = pltpu.PrefetchScalarGridSpec(
        num_scalar_prefetch=5,
        grid=(max_steps,),
        in_specs=[
            pl.BlockSpec((_BM, d), lambda i, em, tm, vm, se, ee: (tm[i], 0)),
            pl.BlockSpec((1, d, h2), lambda i, em, tm, vm, se, ee: (em[i], 0, 0)),
            pl.BlockSpec((1, h, d), lambda i, em, tm, vm, se, ee: (em[i], 0, 0)),
            pl.BlockSpec((_BM, 1), lambda i, em, tm, vm, se, ee: (tm[i], 0)),
        ],
        out_specs=pl.BlockSpec((_BM, d), lambda i, em, tm, vm, se, ee: (tm[i], 0)),
    )
    return pl.pallas_call(
        _gmm_body,
        grid_spec=grid_spec,
        out_shape=jax.ShapeDtypeStruct((tk, d), jnp.float32),
        compiler_params=pltpu.CompilerParams(
            dimension_semantics=("arbitrary",),
            vmem_limit_bytes=100 * 1024 * 1024,
        ),
    )(
        e_of,
        t_of,
        valid,
        starts.astype(jnp.int32),
        ends.astype(jnp.int32),
        x_sorted,
        w_in,
        w_out,
        gates_sorted[:, None],
    )


@jax.jit
def kernel(x, w_router, w_in, w_out, bias):
    bsz, length, d = x.shape
    e_num = w_router.shape[1]
    k = 2
    xf = x.reshape(-1, d)
    t = xf.shape[0]

    logits = (xf @ w_router).astype(jnp.float32)
    top_k_logits, top_k_indices = jax.lax.top_k(logits, k)
    top_k_gates = jax.nn.softmax(top_k_logits, axis=1).astype(x.dtype)
    probs = jax.nn.softmax(logits, axis=1)
    probs_sum = probs.sum(axis=0)
    flat_experts = top_k_indices.reshape(-1)
    flat_gates = top_k_gates.reshape(-1)
    oh = (flat_experts[:, None] == jnp.arange(e_num, dtype=jnp.int32)[None, :])
    freq = (oh & (flat_gates > 0)[:, None]).astype(jnp.float32).sum(axis=0)
    lsesq = (jax.nn.logsumexp(logits, axis=-1) ** 2).sum()
    probs_normalized = probs_sum / jnp.sum(probs_sum)
    freq_normalized = freq / jnp.sum(freq)
    switchloss = e_num * (probs_normalized * freq_normalized).sum()
    zloss = lsesq / t
    loss = switchloss + 0.1 * zloss

    cc = jnp.cumsum(oh.astype(jnp.int32), axis=0)
    counts = cc[-1]
    ends = jnp.cumsum(counts).astype(jnp.int32)
    starts = ends - counts
    rank = starts[flat_experts] + jnp.take_along_axis(
        cc, flat_experts[:, None], axis=1
    )[:, 0] - 1
    ise = jnp.zeros((t * k,), jnp.int32).at[rank].set(
        jnp.arange(t * k, dtype=jnp.int32), mode="drop", unique_indices=True
    )
    batch_index = ise // k
    gates_sorted = flat_gates[ise]

    x_sorted = xf[batch_index]
    out_w = _grouped_ffn(x_sorted, w_in, w_out, gates_sorted, starts, ends)

    y = out_w[rank.reshape(t, k)].sum(axis=1)
    y = y.reshape(bsz, length, d) + bias
    return (y, loss)


# speedup vs baseline: 1.2822x; 1.2822x over previous
import functools

import jax
import jax.numpy as jnp
from jax import lax
from jax.experimental import pallas as pl
from jax.experimental.pallas import tpu as pltpu
from jax.experimental.pallas import tpu_sc as plsc

_BM = 512


def _gmm_body(em, tm, vm, se, ee, x_ref, win_ref, wout_ref, gates_ref, out_ref):
    i = pl.program_id(0)
    e = em[i]
    t = tm[i]
    valid = vm[i]
    bm, d = x_ref.shape
    h2 = win_ref.shape[2]
    h = h2 // 2

    @pl.when(valid == 1)
    def _():
        rows = t * bm + jax.lax.broadcasted_iota(jnp.int32, (bm, 1), 0)
        mask = (rows >= se[e]) & (rows < ee[e])
        hh = jnp.dot(x_ref[...], win_ref[0], preferred_element_type=jnp.float32)
        h1 = hh[:, :h]
        hg = hh[:, h:]
        act = h1 * jax.nn.sigmoid(h1) * hg
        o = jnp.dot(act, wout_ref[0], preferred_element_type=jnp.float32)
        o = o * gates_ref[...]
        out_ref[...] = jnp.where(mask, o, out_ref[...])


def _grouped_ffn(x_sorted, w_in, w_out, gates_sorted, starts, ends):
    tk, d = x_sorted.shape
    e_num, _, h2 = w_in.shape
    h = h2 // 2
    m_tiles = tk // _BM
    max_steps = m_tiles + e_num - 1

    counts = ends - starts
    tile_lo = starts // _BM
    tile_hi = (ends + _BM - 1) // _BM
    ntiles = jnp.where(counts > 0, tile_hi - tile_lo, 0)
    cum = jnp.cumsum(ntiles)
    total = cum[-1]
    first_step = cum - ntiles

    steps = jnp.arange(max_steps, dtype=jnp.int32)
    e_of = jnp.searchsorted(cum, steps, side="right").astype(jnp.int32)
    valid = (steps < total).astype(jnp.int32)
    e_last = jnp.searchsorted(cum, total - 1, side="right").astype(jnp.int32)
    e_of = jnp.where(valid == 1, jnp.minimum(e_of, e_num - 1), e_last)
    t_of = jnp.where(
        valid == 1,
        tile_lo[e_of] + steps - first_step[e_of],
        m_tiles - 1,
    ).astype(jnp.int32)

    grid_spec = pltpu.PrefetchScalarGridSpec(
        num_scalar_prefetch=5,
        grid=(max_steps,),
        in_specs=[
            pl.BlockSpec((_BM, d), lambda i, em, tm, vm, se, ee: (tm[i], 0)),
            pl.BlockSpec((1, d, h2), lambda i, em, tm, vm, se, ee: (em[i], 0, 0)),
            pl.BlockSpec((1, h, d), lambda i, em, tm, vm, se, ee: (em[i], 0, 0)),
            pl.BlockSpec((_BM, 1), lambda i, em, tm, vm, se, ee: (tm[i], 0)),
        ],
        out_specs=pl.BlockSpec((_BM, d), lambda i, em, tm, vm, se, ee: (tm[i], 0)),
    )
    return pl.pallas_call(
        _gmm_body,
        grid_spec=grid_spec,
        out_shape=jax.ShapeDtypeStruct((tk, d), jnp.float32),
        compiler_params=pltpu.CompilerParams(
            dimension_semantics=("arbitrary",),
            vmem_limit_bytes=100 * 1024 * 1024,
        ),
    )(
        e_of,
        t_of,
        valid,
        starts.astype(jnp.int32),
        ends.astype(jnp.int32),
        x_sorted,
        w_in,
        w_out,
        gates_sorted[:, None],
    )


def _sc_dispatch_gather(xf, batch_index):
    t, d = xf.shape
    tk = batch_index.shape[0]
    info = plsc.get_sparse_core_info()
    nw = info.num_cores * info.num_subcores
    bpw = tk // nw
    ch = 32
    nch = bpw // ch
    mesh = plsc.VectorSubcoreMesh(core_axis_name="c", subcore_axis_name="s")

    @functools.partial(
        pl.kernel,
        mesh=mesh,
        out_type=jax.ShapeDtypeStruct((tk, d), jnp.float32),
        scratch_types=[
            pltpu.VMEM((bpw,), jnp.int32),
            pltpu.VMEM((ch, d), jnp.float32),
            pltpu.VMEM((ch, d), jnp.float32),
            pltpu.SemaphoreType.DMA,
            pltpu.SemaphoreType.DMA,
        ],
    )
    def k(xf_hbm, idx_hbm, out_hbm, idx_v, buf0, buf1, sem0, sem1):
        wid = lax.axis_index("s") * info.num_cores + lax.axis_index("c")
        base = wid * bpw
        pltpu.sync_copy(idx_hbm.at[pl.ds(base, bpw)], idx_v)
        bufs = (buf0, buf1)
        sems = (sem0, sem1)
        cps = [None, None]
        for c in range(nch):
            cps[c % 2] = pltpu.async_copy(
                xf_hbm.at[idx_v.at[pl.ds(c * ch, ch)]], bufs[c % 2], sems[c % 2]
            )
            if c > 0:
                cps[(c - 1) % 2].wait()
                pltpu.sync_copy(
                    bufs[(c - 1) % 2], out_hbm.at[pl.ds(base + (c - 1) * ch, ch)]
                )
        cps[(nch - 1) % 2].wait()
        pltpu.sync_copy(
            bufs[(nch - 1) % 2], out_hbm.at[pl.ds(base + (nch - 1) * ch, ch)]
        )

    return k(xf, batch_index)


def _sc_combine(out_w, rank_even, rank_odd, bias):
    tk, d = out_w.shape
    t = rank_even.shape[0]
    info = plsc.get_sparse_core_info()
    nl = info.num_lanes
    nw = info.num_cores * info.num_subcores
    tpw = t // nw
    ch = 32
    nch = tpw // ch
    mesh = plsc.VectorSubcoreMesh(core_axis_name="c", subcore_axis_name="s")

    @functools.partial(
        pl.kernel,
        mesh=mesh,
        out_type=jax.ShapeDtypeStruct((t, d), jnp.float32),
        scratch_types=[
            pltpu.VMEM((tpw,), jnp.int32),
            pltpu.VMEM((tpw,), jnp.int32),
            pltpu.VMEM((d,), jnp.float32),
            pltpu.VMEM((ch, d), jnp.float32),
            pltpu.VMEM((ch, d), jnp.float32),
            pltpu.SemaphoreType.DMA,
            pltpu.SemaphoreType.DMA,
        ],
    )
    def k(ow_hbm, re_hbm, ro_hbm, b_hbm, y_hbm, re_v, ro_v, b_v, bufa, bufb,
          sema, semb):
        wid = lax.axis_index("s") * info.num_cores + lax.axis_index("c")
        base = wid * tpw
        pltpu.sync_copy(re_hbm.at[pl.ds(base, tpw)], re_v)
        pltpu.sync_copy(ro_hbm.at[pl.ds(base, tpw)], ro_v)
        pltpu.sync_copy(b_hbm, b_v)
        for c in range(nch):
            cpa = pltpu.async_copy(
                ow_hbm.at[re_v.at[pl.ds(c * ch, ch)]], bufa, sema
            )
            cpb = pltpu.async_copy(
                ow_hbm.at[ro_v.at[pl.ds(c * ch, ch)]], bufb, semb
            )
            cpa.wait()
            cpb.wait()

            def body(i, _):
                r = i // (d // nl)
                l = (i % (d // nl)) * nl
                bufa[r, pl.ds(l, nl)] = (
                    bufa[r, pl.ds(l, nl)]
                    + bufb[r, pl.ds(l, nl)]
                    + b_v[pl.ds(l, nl)]
                )
                return 0

            lax.fori_loop(0, ch * (d // nl), body, 0)
            pltpu.sync_copy(bufa, y_hbm.at[pl.ds(base + c * ch, ch)])

    return k(out_w, rank_even, rank_odd, bias)


@jax.jit
def kernel(x, w_router, w_in, w_out, bias):
    bsz, length, d = x.shape
    e_num = w_router.shape[1]
    k = 2
    xf = x.reshape(-1, d)
    t = xf.shape[0]

    logits = (xf @ w_router).astype(jnp.float32)
    top_k_logits, top_k_indices = jax.lax.top_k(logits, k)
    top_k_gates = jax.nn.softmax(top_k_logits, axis=1).astype(x.dtype)
    probs = jax.nn.softmax(logits, axis=1)
    probs_sum = probs.sum(axis=0)
    flat_experts = top_k_indices.reshape(-1)
    flat_gates = top_k_gates.reshape(-1)
    freq = jnp.zeros((e_num,), jnp.float32).at[flat_experts].add(
        (flat_gates > 0).astype(jnp.float32)
    )
    lsesq = (jax.nn.logsumexp(logits, axis=-1) ** 2).sum()
    probs_normalized = probs_sum / jnp.sum(probs_sum)
    freq_normalized = freq / jnp.sum(freq)
    switchloss = e_num * (probs_normalized * freq_normalized).sum()
    zloss = lsesq / t
    loss = switchloss + 0.1 * zloss

    ise = jnp.argsort(flat_experts)
    batch_index = (ise // k).astype(jnp.int32)
    gates_sorted = flat_gates[ise]
    rank = jnp.zeros((t * k,), jnp.int32).at[ise].set(
        jnp.arange(t * k, dtype=jnp.int32), mode="drop", unique_indices=True
    )

    counts = jnp.zeros((e_num,), jnp.int32).at[flat_experts].add(1)
    ends = jnp.cumsum(counts).astype(jnp.int32)
    starts = ends - counts

    x_sorted = _sc_dispatch_gather(xf, batch_index)
    out_w = _grouped_ffn(x_sorted, w_in, w_out, gates_sorted, starts, ends)

    y = _sc_combine(out_w, rank[0::2], rank[1::2], bias)
    y = y.reshape(bsz, length, d)
    return (y, loss)
